# baseline (speedup 1.0000x reference)
import jax
import jax.numpy as jnp
from jax import lax
from jax.experimental import pallas as pl
from jax.experimental.pallas import tpu as pltpu


def _body(mw_ref, w0_ref, w1_ref, x_hbm, o_hbm, xbuf, obuf, lsem, ssem):
    B, L, C = x_hbm.shape
    w0 = w0_ref[...]
    w1 = w1_ref[...]

    def start_load(b, slot):
        pltpu.make_async_copy(x_hbm.at[b], xbuf.at[slot], lsem.at[slot]).start()

    def wait_load(b, slot):
        pltpu.make_async_copy(x_hbm.at[b], xbuf.at[slot], lsem.at[slot]).wait()

    def start_store(b, slot):
        pltpu.make_async_copy(obuf.at[slot], o_hbm.at[b], ssem.at[slot]).start()

    def wait_store(b, slot):
        pltpu.make_async_copy(obuf.at[slot], o_hbm.at[b], ssem.at[slot]).wait()

    def compute(b, slot, oslot):
        xb = xbuf[slot]
        mrow = mw_ref[b]
        sums = lax.dot_general(mrow, xb, (((1,), (0,)), ((), ())),
                               preferred_element_type=jnp.float32)
        cnt = jnp.sum(mrow)
        mean = sums / jnp.maximum(cnt, 1.0)
        minf = (mrow.reshape(L, 1) - 1.0) * jnp.float32(1e30)
        mx = jnp.max(xb + minf, axis=0, keepdims=True)

        def mlp(v):
            h = lax.dot_general(v, w0, (((1,), (1,)), ((), ())),
                                preferred_element_type=jnp.float32)
            h = jnp.maximum(h, 0.0)
            return lax.dot_general(h, w1, (((1,), (1,)), ((), ())),
                                   preferred_element_type=jnp.float32)

        a = jax.nn.sigmoid(mlp(mean) + mlp(mx))
        obuf[oslot] = xb * a

    start_load(0, 0)
    start_load(1, 1)
    start_load(2, 2)
    for b in range(B):
        if b + 3 < B:
            start_load(b + 3, (b + 3) % 4)
        wait_load(b, b % 4)
        if b >= 2:
            wait_store(b - 2, b % 2)
        compute(b, b % 4, b % 2)
        start_store(b, b % 2)
    wait_store(B - 2, 0)
    wait_store(B - 1, 1)


def kernel(x, attention_mask, W0, W1):
    B, L, C = x.shape
    mw = attention_mask.astype(jnp.float32).reshape(B, 1, L)
    return pl.pallas_call(
        _body,
        in_specs=[
            pl.BlockSpec(memory_space=pltpu.MemorySpace.VMEM),
            pl.BlockSpec(memory_space=pltpu.MemorySpace.VMEM),
            pl.BlockSpec(memory_space=pltpu.MemorySpace.VMEM),
            pl.BlockSpec(memory_space=pl.ANY),
        ],
        out_specs=pl.BlockSpec(memory_space=pl.ANY),
        out_shape=jax.ShapeDtypeStruct(x.shape, x.dtype),
        scratch_shapes=[
            pltpu.VMEM((4, L, C), jnp.float32),
            pltpu.VMEM((2, L, C), jnp.float32),
            pltpu.SemaphoreType.DMA((4,)),
            pltpu.SemaphoreType.DMA((2,)),
        ],
    )(mw, W0, W1, x)
